# baseline (speedup 1.0000x reference)
import functools

import jax
import jax.numpy as jnp
from jax import lax
from jax.experimental import pallas as pl
from jax.experimental.pallas import tpu as pltpu
from jax.experimental.pallas import tpu_sc as plsc

EMB = 64
B = 4096
T = 200
NC = 2
NS = 16
NW = NC * NS
CHUNK = B // NW
LANES = 16

_mesh = plsc.VectorSubcoreMesh(core_axis_name="c", subcore_axis_name="s")


@functools.partial(
    pl.kernel,
    out_type=jax.ShapeDtypeStruct((T, EMB, B), jnp.float32),
    mesh=_mesh,
    scratch_types=[
        pltpu.VMEM((T, CHUNK), jnp.int32),
        pltpu.VMEM((2, CHUNK, EMB), jnp.float32),
        pltpu.VMEM((2, EMB, CHUNK), jnp.float32),
        pltpu.SemaphoreType.DMA,
        pltpu.SemaphoreType.DMA,
    ],
    compiler_params=pltpu.CompilerParams(use_tc_tiling_on_sc=False, needs_layout_passes=False),
)
def _gather(table_hbm, toks_hbm, out_hbm, idx_v, rows_v, trans_v, gsem, psem):
    wid = lax.axis_index("s") * NC + lax.axis_index("c")

    def start_gather(t, bf):
        pltpu.async_copy(table_hbm.at[idx_v.at[t]], rows_v.at[bf], gsem)

    def wait_gather():
        pltpu.make_async_copy(
            table_hbm.at[idx_v.at[0]], rows_v.at[0], gsem).wait()

    def start_put(t, bf):
        pltpu.async_copy(
            trans_v.at[bf],
            out_hbm.at[t].at[:, pl.ds(wid * CHUNK, CHUNK)],
            psem)

    def wait_put():
        pltpu.make_async_copy(
            trans_v.at[0],
            out_hbm.at[0].at[:, pl.ds(wid * CHUNK, CHUNK)],
            psem).wait()

    lane = lax.iota(jnp.int32, LANES)
    col_ids = [lane + j * LANES for j in range(CHUNK // LANES)]

    def transpose(bf):
        src = rows_v.at[bf]
        dst = trans_v.at[bf]

        @pl.loop(0, EMB)
        def _(e):
            e_vec = jnp.full((LANES,), 0, jnp.int32) + e
            drow = dst.at[e]
            for j in range(CHUNK // LANES):
                vals = plsc.load_gather(src, [col_ids[j], e_vec])
                drow[pl.ds(j * LANES, LANES)] = vals

    pltpu.sync_copy(toks_hbm.at[:, wid, :], idx_v)

    start_gather(0, 0)

    @pl.loop(0, T)
    def _(t):
        bf = lax.rem(t, 2)
        wait_gather()

        @pl.when(t < T - 1)
        def _():
            start_gather(t + 1, 1 - bf)

        @pl.when(t >= 2)
        def _():
            wait_put()

        transpose(bf)
        start_put(t, bf)

    wait_put()
    wait_put()


def kernel(toks, table):
    toks_nat = toks.T.reshape(T, NW, CHUNK)
    out = _gather(table, toks_nat)
    return out.transpose(2, 0, 1)


# speedup vs baseline: 1.6448x; 1.6448x over previous
import functools

import jax
import jax.numpy as jnp
from jax import lax
from jax.experimental import pallas as pl
from jax.experimental.pallas import tpu as pltpu
from jax.experimental.pallas import tpu_sc as plsc

EMB = 64
B = 4096
T = 200
NC = 2
NS = 16
NW = NC * NS
NBC = 8
BCH = B // NBC
NTG = NW // NBC
TG = T // NTG
KS = BCH // 128
LANES = 16
PITCH = BCH + 1

_mesh = plsc.VectorSubcoreMesh(core_axis_name="c", subcore_axis_name="s")


@functools.partial(
    pl.kernel,
    out_type=jax.ShapeDtypeStruct((T, EMB, B), jnp.float32),
    mesh=_mesh,
    scratch_types=[
        pltpu.VMEM((3, BCH), jnp.int32),
        pltpu.VMEM((2, BCH, EMB), jnp.float32),
        pltpu.VMEM((EMB, PITCH), jnp.float32),
        pltpu.SemaphoreType.DMA,
        pltpu.SemaphoreType.DMA,
        pltpu.SemaphoreType.DMA,
    ],
    compiler_params=pltpu.CompilerParams(
        use_tc_tiling_on_sc=False, needs_layout_passes=False),
)
def _gather(table_hbm, toks_hbm, out_hbm, idx_v, rows_v, trans_v,
            isem, gsem, psem):
    wid = lax.axis_index("s") * NC + lax.axis_index("c")
    bc = lax.rem(wid, NBC)
    tg = wid // NBC
    t0 = tg * TG

    lane = lax.iota(jnp.int32, LANES)
    e_ids = [lane + k * LANES for k in range(EMB // LANES)]

    def idx_load(t, slot):
        pltpu.async_copy(toks_hbm.at[t0 + t, bc], idx_v.at[slot], isem)

    def idx_wait():
        pltpu.make_async_copy(toks_hbm.at[0, 0], idx_v.at[0], isem).wait()

    def start_gather(t, rb):
        slot = lax.rem(t, 3)
        for k in range(KS):
            pltpu.async_copy(
                table_hbm.at[idx_v.at[slot].at[pl.ds(k * 128, 128)]],
                rows_v.at[rb].at[pl.ds(k * 128, 128)],
                gsem)

    def wait_gather():
        for k in range(KS):
            pltpu.make_async_copy(
                table_hbm.at[idx_v.at[0].at[pl.ds(0, 128)]],
                rows_v.at[0].at[pl.ds(0, 128)],
                gsem).wait()

    def start_put(t):
        pltpu.async_copy(
            trans_v.at[:, pl.ds(0, BCH)],
            out_hbm.at[t0 + t].at[:, pl.ds(bc * BCH, BCH)],
            psem)

    def wait_put():
        pltpu.make_async_copy(
            trans_v.at[:, pl.ds(0, BCH)],
            out_hbm.at[0].at[:, pl.ds(0, BCH)],
            psem).wait()

    def transpose(rb):
        src = rows_v.at[rb]

        @pl.loop(0, BCH, step=4)
        def _(b0):
            for u in range(4):
                b = b0 + u
                bv = jnp.full((LANES,), 0, jnp.int32) + b
                for k in range(EMB // LANES):
                    vals = src[b, pl.ds(k * LANES, LANES)]
                    plsc.store_scatter(trans_v, [e_ids[k], bv], vals)

    idx_load(0, 0)
    idx_wait()
    idx_load(1, 1)
    start_gather(0, 0)

    @pl.loop(0, TG)
    def _(t):
        rb = lax.rem(t, 2)

        @pl.when(t < TG - 1)
        def _():
            idx_wait()
            start_gather(t + 1, 1 - rb)

            @pl.when(t < TG - 2)
            def _():
                idx_load(t + 2, lax.rem(t + 2, 3))

        wait_gather()

        @pl.when(t >= 1)
        def _():
            wait_put()

        transpose(rb)
        start_put(t)

    wait_put()


def kernel(toks, table):
    toks_nat = toks.T.reshape(T, NBC, BCH)
    out = _gather(table, toks_nat)
    return out.transpose(2, 0, 1)
